# baseline (speedup 1.0000x reference)
import functools

import jax
import jax.numpy as jnp
from jax import lax
from jax.experimental import pallas as pl
from jax.experimental.pallas import tpu as pltpu
from jax.experimental.pallas import tpu_sc as plsc

VOCAB = 49408
HIDDEN = 768
MAX_POS = 77
BATCH = 1024
SEQ = 77

NC = 2
NS = 16
NW = NC * NS

B = BATCH * SEQ
BPW = BATCH // NW
HALF = 2
CB = BPW // HALF
NCHUNKS = SEQ * HALF
LANES = 16
NVEC = HIDDEN // LANES
NBUF = 4


def _body(table_hbm, idx_hbm, pos_hbm, out_hbm, idx_v, pos_v, buf, gsem, ssem):
    wid = lax.axis_index("s") * NC + lax.axis_index("c")
    col0 = wid * BPW

    pltpu.sync_copy(idx_hbm.at[wid], idx_v)
    pltpu.sync_copy(pos_hbm, pos_v)

    def _dst(q):
        row0 = lax.div(q, HALF) * BATCH + col0 + lax.rem(q, HALF) * CB
        return out_hbm.at[pl.ds(row0, CB)]

    def gather_start(q):
        m = lax.rem(q, NBUF)
        pltpu.async_copy(table_hbm.at[idx_v.at[q]], buf.at[m], gsem)

    def gather_wait(q):
        m = lax.rem(q, NBUF)
        pltpu.make_async_copy(table_hbm.at[idx_v.at[q]], buf.at[m], gsem).wait()

    def scatter_start(q):
        m = lax.rem(q, NBUF)
        pltpu.async_copy(buf.at[m], _dst(q), ssem)

    def scatter_wait(q):
        m = lax.rem(q, NBUF)
        pltpu.make_async_copy(buf.at[m], _dst(q), ssem).wait()

    for q in range(NBUF - 1):
        gather_start(q)

    def chunk_body(q, _):
        @pl.when(q >= 1)
        def _():
            scatter_wait(q - 1)

        @pl.when(q + NBUF - 1 < NCHUNKS)
        def _():
            gather_start(q + NBUF - 1)

        gather_wait(q)
        m = lax.rem(q, NBUF)
        p = lax.div(q, HALF)

        def col_body(j, _):
            sl = pl.ds(j * LANES, LANES)
            pv = pos_v[p, sl]
            for b in range(CB):
                buf[m, b, sl] += pv
            return 0

        lax.fori_loop(0, NVEC, col_body, 0)

        scatter_start(q)
        return 0

    lax.fori_loop(0, NCHUNKS, chunk_body, 0)
    scatter_wait(NCHUNKS - 1)


_sc_call = functools.partial(
    pl.kernel,
    out_type=jax.ShapeDtypeStruct((B, HIDDEN), jnp.float32),
    mesh=plsc.VectorSubcoreMesh(
        core_axis_name="c", subcore_axis_name="s", num_cores=NC, num_subcores=NS
    ),
    scratch_types=[
        pltpu.VMEM((NCHUNKS, CB), jnp.int32),
        pltpu.VMEM((MAX_POS, HIDDEN), jnp.float32),
        pltpu.VMEM((NBUF, CB, HIDDEN), jnp.float32),
        pltpu.SemaphoreType.DMA,
        pltpu.SemaphoreType.DMA,
    ],
)(_body)


@jax.jit
def kernel(input_ids, token_table, pos_table):
    ids = input_ids.astype(jnp.int32).reshape(NW, HALF, CB, SEQ)
    ids = ids.transpose(0, 3, 1, 2).reshape(NW, NCHUNKS, CB)
    out = _sc_call(token_table, ids, pos_table)
    return out.reshape(SEQ, BATCH, HIDDEN).transpose(1, 0, 2)


# speedup vs baseline: 1.0805x; 1.0805x over previous
import functools

import jax
import jax.numpy as jnp
from jax import lax
from jax.experimental import pallas as pl
from jax.experimental.pallas import tpu as pltpu
from jax.experimental.pallas import tpu_sc as plsc

VOCAB = 49408
HIDDEN = 768
MAX_POS = 77
BATCH = 1024
SEQ = 77

NC = 2
NS = 16
NW = NC * NS

B = BATCH * SEQ
BPW = BATCH // NW
LANES = 16
NVEC = HIDDEN // LANES
NBUF = 4


def _body(table_hbm, idx_hbm, pos_hbm, out_hbm,
          idx_v, pos_v, buf, gsem, psem, ssem):
    wid = lax.axis_index("s") * NC + lax.axis_index("c")
    col0 = wid * BPW

    pltpu.sync_copy(idx_hbm.at[wid], idx_v)

    def gather_start(p):
        m = lax.rem(p, NBUF)
        pltpu.async_copy(table_hbm.at[idx_v.at[p]], buf.at[m], gsem)
        pltpu.async_copy(pos_hbm.at[p], pos_v.at[m], psem)

    def gather_wait(p):
        m = lax.rem(p, NBUF)
        pltpu.make_async_copy(table_hbm.at[idx_v.at[p]], buf.at[m], gsem).wait()
        pltpu.make_async_copy(pos_hbm.at[p], pos_v.at[m], psem).wait()

    def scatter_start(p):
        m = lax.rem(p, NBUF)
        pltpu.async_copy(buf.at[m], out_hbm.at[pl.ds(p * BATCH + col0, BPW)], ssem)

    def scatter_wait(p):
        m = lax.rem(p, NBUF)
        pltpu.make_async_copy(
            buf.at[m], out_hbm.at[pl.ds(p * BATCH + col0, BPW)], ssem
        ).wait()

    for p in range(NBUF - 1):
        gather_start(p)

    def chunk_body(p, _):
        @pl.when(p >= 1)
        def _():
            scatter_wait(p - 1)

        @pl.when(p + NBUF - 1 < SEQ)
        def _():
            gather_start(p + NBUF - 1)

        gather_wait(p)
        m = lax.rem(p, NBUF)

        def col_body(j, _):
            sl = pl.ds(j * LANES, LANES)
            pv = pos_v[m, 0, sl]
            for b in range(BPW):
                buf[m, b, sl] += pv
            return 0

        lax.fori_loop(0, NVEC, col_body, 0)

        scatter_start(p)
        return 0

    lax.fori_loop(0, SEQ, chunk_body, 0)
    scatter_wait(SEQ - 1)


_sc_call = functools.partial(
    pl.kernel,
    out_type=jax.ShapeDtypeStruct((B, HIDDEN), jnp.float32),
    mesh=plsc.VectorSubcoreMesh(
        core_axis_name="c", subcore_axis_name="s", num_cores=NC, num_subcores=NS
    ),
    scratch_types=[
        pltpu.VMEM((SEQ, BPW), jnp.int32),
        pltpu.VMEM((NBUF, 1, HIDDEN), jnp.float32),
        pltpu.VMEM((NBUF, BPW, HIDDEN), jnp.float32),
        pltpu.SemaphoreType.DMA,
        pltpu.SemaphoreType.DMA,
        pltpu.SemaphoreType.DMA,
    ],
)(_body)


@jax.jit
def kernel(input_ids, token_table, pos_table):
    ids = input_ids.astype(jnp.int32).reshape(NW, BPW, SEQ).transpose(0, 2, 1)
    pos = pos_table.reshape(SEQ, 1, HIDDEN)
    out = _sc_call(token_table, ids, pos)
    return out.reshape(SEQ, BATCH, HIDDEN).transpose(1, 0, 2)


# speedup vs baseline: 1.2028x; 1.1132x over previous
import functools

import jax
import jax.numpy as jnp
from jax import lax
from jax.experimental import pallas as pl
from jax.experimental.pallas import tpu as pltpu
from jax.experimental.pallas import tpu_sc as plsc

VOCAB = 49408
HIDDEN = 768
MAX_POS = 77
BATCH = 1024
SEQ = 77

NC = 2
NS = 16
NW = NC * NS

B = BATCH * SEQ
BPW = BATCH // NW
LANES = 16
NVEC = HIDDEN // LANES
NBUF = 4


def _body(table_hbm, idx_hbm, pos_hbm, out_hbm,
          idx_v, pos_v, buf, gsem, psem, ssem):
    wid = lax.axis_index("s") * NC + lax.axis_index("c")
    col0 = wid * BPW

    pltpu.sync_copy(idx_hbm.at[wid], idx_v)

    def gather_start(p):
        m = lax.rem(p, NBUF)
        pltpu.async_copy(table_hbm.at[idx_v.at[p]], buf.at[m], gsem)
        pltpu.async_copy(pos_hbm.at[p], pos_v.at[m], psem)

    def gather_wait(p):
        m = lax.rem(p, NBUF)
        pltpu.make_async_copy(table_hbm.at[idx_v.at[p]], buf.at[m], gsem).wait()
        pltpu.make_async_copy(pos_hbm.at[p], pos_v.at[m], psem).wait()

    HB = BPW // 2

    def scatter_start(p, h):
        m = lax.rem(p, NBUF)
        pltpu.async_copy(
            buf.at[m, pl.ds(h * HB, HB)],
            out_hbm.at[pl.ds(p * BATCH + col0 + h * HB, HB)], ssem)

    def scatter_wait(p):
        m = lax.rem(p, NBUF)
        for h in range(2):
            pltpu.make_async_copy(
                buf.at[m, pl.ds(h * HB, HB)],
                out_hbm.at[pl.ds(p * BATCH + col0 + h * HB, HB)], ssem
            ).wait()

    for p in range(NBUF - 1):
        gather_start(p)

    def chunk_body(p, _):
        @pl.when(p >= 1)
        def _():
            scatter_wait(p - 1)

        @pl.when(p + NBUF - 1 < SEQ)
        def _():
            gather_start(p + NBUF - 1)

        gather_wait(p)
        m = lax.rem(p, NBUF)

        def make_col_body(h):
            def col_body(j, _):
                sl = pl.ds(j * LANES, LANES)
                pv = pos_v[m, 0, sl]
                for b in range(h * HB, (h + 1) * HB):
                    buf[m, b, sl] += pv
                return 0
            return col_body

        lax.fori_loop(0, NVEC, make_col_body(0), 0)
        scatter_start(p, 0)
        lax.fori_loop(0, NVEC, make_col_body(1), 0)
        scatter_start(p, 1)
        return 0

    lax.fori_loop(0, SEQ, chunk_body, 0)
    scatter_wait(SEQ - 1)


_sc_call = functools.partial(
    pl.kernel,
    out_type=jax.ShapeDtypeStruct((B, HIDDEN), jnp.float32),
    mesh=plsc.VectorSubcoreMesh(
        core_axis_name="c", subcore_axis_name="s", num_cores=NC, num_subcores=NS
    ),
    scratch_types=[
        pltpu.VMEM((SEQ, BPW), jnp.int32),
        pltpu.VMEM((NBUF, 1, HIDDEN), jnp.float32),
        pltpu.VMEM((NBUF, BPW, HIDDEN), jnp.float32),
        pltpu.SemaphoreType.DMA,
        pltpu.SemaphoreType.DMA,
        pltpu.SemaphoreType.DMA,
    ],
)(_body)


@jax.jit
def kernel(input_ids, token_table, pos_table):
    ids = input_ids.astype(jnp.int32).reshape(NW, BPW, SEQ).transpose(0, 2, 1)
    pos = pos_table.reshape(SEQ, 1, HIDDEN)
    out = _sc_call(token_table, ids, pos)
    return out.reshape(SEQ, BATCH, HIDDEN).transpose(1, 0, 2)
